# Initial kernel scaffold; baseline (speedup 1.0000x reference)
import functools

import jax
import jax.numpy as jnp
from jax import lax
from jax.experimental import pallas as pl
from jax.experimental.pallas import tpu as pltpu
from jax.experimental.pallas import tpu_sc as plsc

VOCAB = 1000000
MAX_COUNT = 100
EMB = 32
B = 16384
L = 50
N = B * L

NUM_CORES = 2
NUM_SUBCORES = 16
NW = NUM_CORES * NUM_SUBCORES
ROWS_PER_W = N // NW
IDXV = 128
CHUNK = 1024
JROWS = CHUNK // IDXV
N_CHUNKS = ROWS_PER_W // CHUNK
IDX_ROWS_PER_W = ROWS_PER_W // IDXV


def _make_kernel():
    mesh = plsc.VectorSubcoreMesh(core_axis_name="c", subcore_axis_name="s")

    @functools.partial(
        pl.kernel,
        mesh=mesh,
        out_type=jax.ShapeDtypeStruct((N, EMB), jnp.float32),
        scratch_types=[
            pltpu.VMEM((JROWS, IDXV), jnp.int32),
            pltpu.VMEM((JROWS, IDXV), jnp.int32),
            pltpu.VMEM((CHUNK, EMB), jnp.float32),
            pltpu.VMEM((CHUNK, EMB), jnp.float32),
            pltpu.SemaphoreType.DMA,
            pltpu.SemaphoreType.DMA,
        ],
    )
    def emb_kernel(didx_hbm, cidx_hbm, dtab_hbm, ctab_hbm, out_hbm,
                   didx_v, cidx_v, drows_v, crows_v, dsem, csem):
        wid = lax.axis_index("s") * NUM_CORES + lax.axis_index("c")
        idx_row0 = wid * IDX_ROWS_PER_W
        out_row0 = wid * ROWS_PER_W

        def chunk_body(t, carry):
            base_idx_row = idx_row0 + t * JROWS
            pltpu.sync_copy(didx_hbm.at[pl.ds(base_idx_row, JROWS)], didx_v)
            pltpu.sync_copy(cidx_hbm.at[pl.ds(base_idx_row, JROWS)], cidx_v)
            dcopies = []
            ccopies = []
            for j in range(JROWS):
                dcopies.append(pltpu.async_copy(
                    dtab_hbm.at[didx_v.at[j]],
                    drows_v.at[pl.ds(j * IDXV, IDXV)], dsem))
                ccopies.append(pltpu.async_copy(
                    ctab_hbm.at[cidx_v.at[j]],
                    crows_v.at[pl.ds(j * IDXV, IDXV)], csem))
            for c in dcopies:
                c.wait()
            for c in ccopies:
                c.wait()

            def add_body(r, c2):
                drows_v[r, pl.ds(0, 16)] = (
                    drows_v[r, pl.ds(0, 16)] + crows_v[r, pl.ds(0, 16)])
                drows_v[r, pl.ds(16, 16)] = (
                    drows_v[r, pl.ds(16, 16)] + crows_v[r, pl.ds(16, 16)])
                return c2

            lax.fori_loop(0, CHUNK, add_body, 0, unroll=4)

            pltpu.sync_copy(drows_v, out_hbm.at[pl.ds(out_row0 + t * CHUNK, CHUNK)])
            return carry

        lax.fori_loop(0, N_CHUNKS, chunk_body, 0)

    return emb_kernel


_emb_kernel = _make_kernel()


def kernel(diseases, counts, disease_table, counts_table):
    didx = diseases.astype(jnp.int32).reshape(N // IDXV, IDXV)
    cidx = counts.astype(jnp.int32).reshape(N // IDXV, IDXV)
    out = _emb_kernel(didx, cidx, disease_table, counts_table)
    return out.reshape(B, L, EMB)


# speedup vs baseline: 1.7980x; 1.7980x over previous
import functools

import jax
import jax.numpy as jnp
from jax import lax
from jax.experimental import pallas as pl
from jax.experimental.pallas import tpu as pltpu
from jax.experimental.pallas import tpu_sc as plsc

VOCAB = 1000000
MAX_COUNT = 100
EMB = 32
B = 16384
L = 50
N = B * L

NUM_CORES = 2
NUM_SUBCORES = 16
NW = NUM_CORES * NUM_SUBCORES
ROWS_PER_W = N // NW
IDXV = 128
CHUNK = 512
JROWS = CHUNK // IDXV
N_CHUNKS = ROWS_PER_W // CHUNK
N_PAIRS = N_CHUNKS // 2
IDX_ROWS_PER_W = ROWS_PER_W // IDXV
HALF = 16


def _make_kernel():
    mesh = plsc.VectorSubcoreMesh(core_axis_name="c", subcore_axis_name="s")

    @functools.partial(
        pl.kernel,
        mesh=mesh,
        out_type=jax.ShapeDtypeStruct((N, EMB), jnp.float32),
        compiler_params=pltpu.CompilerParams(
            use_tc_tiling_on_sc=False, needs_layout_passes=False),
        scratch_types=[
            pltpu.VMEM((IDX_ROWS_PER_W, IDXV), jnp.int32),
            pltpu.VMEM((IDX_ROWS_PER_W, IDXV), jnp.int32),
            pltpu.VMEM((MAX_COUNT, EMB), jnp.float32),
            pltpu.VMEM((CHUNK, EMB), jnp.float32),
            pltpu.VMEM((CHUNK, EMB), jnp.float32),
            pltpu.SemaphoreType.DMA,
            pltpu.SemaphoreType.DMA,
            pltpu.SemaphoreType.DMA,
            pltpu.SemaphoreType.DMA,
        ],
    )
    def emb_kernel(didx_hbm, cidx_hbm, dtab_hbm, ctab_hbm, out_hbm,
                   didx_v, cidx_v, ctab_v, dr_a, dr_b,
                   gsem_a, gsem_b, osem_a, osem_b):
        cid = lax.axis_index("c")
        sid = lax.axis_index("s")
        wid = sid * NUM_CORES + cid
        idx_row0 = wid * IDX_ROWS_PER_W
        out_row0 = wid * ROWS_PER_W

        pltpu.sync_copy(didx_hbm.at[pl.ds(idx_row0, IDX_ROWS_PER_W)], didx_v)
        pltpu.sync_copy(cidx_hbm.at[pl.ds(idx_row0, IDX_ROWS_PER_W)], cidx_v)
        pltpu.sync_copy(ctab_hbm, ctab_v)

        def fire_gathers(t, drows, gsem):
            copies = []
            for j in range(JROWS):
                row = t * JROWS + j
                copies.append(pltpu.async_copy(
                    dtab_hbm.at[didx_v.at[row]],
                    drows.at[pl.ds(j * IDXV, IDXV)], gsem))
            return copies

        def alu_add(t, drows):
            lane = lax.iota(jnp.int32, HALF)
            for j in range(JROWS):
                idx_row = t * JROWS + j

                def body(g, carry, j=j, idx_row=idx_row):
                    c16 = cidx_v[idx_row, pl.ds(g * HALF, HALF)]
                    rows = j * IDXV + g * HALF + lane
                    zero = lane - lane
                    for e in range(EMB):
                        vals = plsc.load_gather(ctab_v, [c16, zero + e])
                        plsc.addupdate_scatter(
                            drows, [rows, zero + e], vals)
                    return carry

                lax.fori_loop(0, IDXV // HALF, body, 0)

        def store(t, drows, osem):
            return pltpu.async_copy(
                drows, out_hbm.at[pl.ds(out_row0 + t * CHUNK, CHUNK)], osem)

        def pair_body(i, carry):
            t0 = 2 * i
            t1 = t0 + 1
            g_a = fire_gathers(t0, dr_a, gsem_a)
            g_b = fire_gathers(t1, dr_b, gsem_b)
            for c in g_a:
                c.wait()
            alu_add(t0, dr_a)
            o_a = store(t0, dr_a, osem_a)
            for c in g_b:
                c.wait()
            alu_add(t1, dr_b)
            o_b = store(t1, dr_b, osem_b)
            o_a.wait()
            o_b.wait()
            return carry

        lax.fori_loop(0, N_PAIRS, pair_body, 0)

    return emb_kernel


_emb_kernel = _make_kernel()


def kernel(diseases, counts, disease_table, counts_table):
    didx = diseases.astype(jnp.int32).reshape(N // IDXV, IDXV)
    cidx = counts.astype(jnp.int32).reshape(N // IDXV, IDXV)
    out = _emb_kernel(didx, cidx, disease_table, counts_table)
    return out.reshape(B, L, EMB)


# speedup vs baseline: 2.6261x; 1.4606x over previous
import functools

import jax
import jax.numpy as jnp
from jax import lax
from jax.experimental import pallas as pl
from jax.experimental.pallas import tpu as pltpu
from jax.experimental.pallas import tpu_sc as plsc

VOCAB = 1000000
MAX_COUNT = 100
EMB = 32
B = 16384
L = 50
N = B * L

NUM_CORES = 2
NUM_SUBCORES = 16
NW = NUM_CORES * NUM_SUBCORES
ROWS_PER_W = N // NW
IDXV = 128
CHUNK = 512
JROWS = CHUNK // IDXV
N_CHUNKS = ROWS_PER_W // CHUNK
N_PAIRS = N_CHUNKS // 2
IDX_ROWS_PER_W = ROWS_PER_W // IDXV
HALF = 16


def _make_kernel():
    mesh = plsc.VectorSubcoreMesh(core_axis_name="c", subcore_axis_name="s")

    @functools.partial(
        pl.kernel,
        mesh=mesh,
        out_type=jax.ShapeDtypeStruct((N, EMB), jnp.float32),
        compiler_params=pltpu.CompilerParams(
            use_tc_tiling_on_sc=False, needs_layout_passes=False),
        scratch_types=[
            pltpu.VMEM((IDX_ROWS_PER_W, IDXV), jnp.int32),
            pltpu.VMEM((IDX_ROWS_PER_W, IDXV), jnp.int32),
            pltpu.VMEM((MAX_COUNT, EMB), jnp.float32),
            pltpu.VMEM((CHUNK, EMB), jnp.float32),
            pltpu.VMEM((CHUNK, EMB), jnp.float32),
            pltpu.SemaphoreType.DMA,
            pltpu.SemaphoreType.DMA,
            pltpu.SemaphoreType.DMA,
            pltpu.SemaphoreType.DMA,
        ],
    )
    def emb_kernel(didx_hbm, cidx_hbm, dtab_hbm, ctab_hbm, out_hbm,
                   didx_v, cidx_v, ctab_v, dr_a, dr_b,
                   gsem_a, gsem_b, osem_a, osem_b):
        cid = lax.axis_index("c")
        sid = lax.axis_index("s")
        wid = sid * NUM_CORES + cid
        idx_row0 = wid * IDX_ROWS_PER_W
        out_row0 = wid * ROWS_PER_W

        pltpu.sync_copy(didx_hbm.at[pl.ds(idx_row0, IDX_ROWS_PER_W)], didx_v)
        pltpu.sync_copy(cidx_hbm.at[pl.ds(idx_row0, IDX_ROWS_PER_W)], cidx_v)
        pltpu.sync_copy(ctab_hbm, ctab_v)

        def fire_gathers(t, drows, gsem):
            copies = []
            for j in range(JROWS):
                row = t * JROWS + j
                copies.append(pltpu.async_copy(
                    dtab_hbm.at[didx_v.at[row]],
                    drows.at[pl.ds(j * IDXV, IDXV)], gsem))
            return copies

        def alu_add(t, drows):
            for j in range(JROWS):
                idx_row = t * JROWS + j

                def body(g, carry, j=j, idx_row=idx_row):
                    c16 = cidx_v[idx_row, pl.ds(g * HALF, HALF)]
                    for k in range(HALF):
                        c = c16[k]
                        r = j * IDXV + g * HALF + k
                        plsc.addupdate(
                            drows.at[r, pl.ds(0, HALF)],
                            ctab_v[c, pl.ds(0, HALF)])
                        plsc.addupdate(
                            drows.at[r, pl.ds(HALF, HALF)],
                            ctab_v[c, pl.ds(HALF, HALF)])
                    return carry

                lax.fori_loop(0, IDXV // HALF, body, 0)

        def store(t, drows, osem):
            return pltpu.async_copy(
                drows, out_hbm.at[pl.ds(out_row0 + t * CHUNK, CHUNK)], osem)

        def pair_body(i, carry):
            t0 = 2 * i
            t1 = t0 + 1
            g_a = fire_gathers(t0, dr_a, gsem_a)
            g_b = fire_gathers(t1, dr_b, gsem_b)
            for c in g_a:
                c.wait()
            alu_add(t0, dr_a)
            o_a = store(t0, dr_a, osem_a)
            for c in g_b:
                c.wait()
            alu_add(t1, dr_b)
            o_b = store(t1, dr_b, osem_b)
            o_a.wait()
            o_b.wait()
            return carry

        lax.fori_loop(0, N_PAIRS, pair_body, 0)

    return emb_kernel


_emb_kernel = _make_kernel()


def kernel(diseases, counts, disease_table, counts_table):
    didx = diseases.astype(jnp.int32).reshape(N // IDXV, IDXV)
    cidx = counts.astype(jnp.int32).reshape(N // IDXV, IDXV)
    out = _emb_kernel(didx, cidx, disease_table, counts_table)
    return out.reshape(B, L, EMB)


# speedup vs baseline: 2.6316x; 1.0021x over previous
import functools

import jax
import jax.numpy as jnp
from jax import lax
from jax.experimental import pallas as pl
from jax.experimental.pallas import tpu as pltpu
from jax.experimental.pallas import tpu_sc as plsc

VOCAB = 1000000
MAX_COUNT = 100
EMB = 32
B = 16384
L = 50
N = B * L

NUM_CORES = 2
NUM_SUBCORES = 16
NW = NUM_CORES * NUM_SUBCORES
ROWS_PER_W = N // NW
IDXV = 512
CHUNK = 512
JROWS = CHUNK // IDXV
N_CHUNKS = ROWS_PER_W // CHUNK
N_PAIRS = N_CHUNKS // 2
IDX_ROWS_PER_W = ROWS_PER_W // IDXV
HALF = 16


def _make_kernel():
    mesh = plsc.VectorSubcoreMesh(core_axis_name="c", subcore_axis_name="s")

    @functools.partial(
        pl.kernel,
        mesh=mesh,
        out_type=jax.ShapeDtypeStruct((N, EMB), jnp.float32),
        compiler_params=pltpu.CompilerParams(
            use_tc_tiling_on_sc=False, needs_layout_passes=False),
        scratch_types=[
            pltpu.VMEM((IDX_ROWS_PER_W, IDXV), jnp.int32),
            pltpu.VMEM((IDX_ROWS_PER_W, IDXV), jnp.int32),
            pltpu.VMEM((MAX_COUNT, EMB), jnp.float32),
            pltpu.VMEM((CHUNK, EMB), jnp.float32),
            pltpu.VMEM((CHUNK, EMB), jnp.float32),
            pltpu.SemaphoreType.DMA,
            pltpu.SemaphoreType.DMA,
            pltpu.SemaphoreType.DMA,
            pltpu.SemaphoreType.DMA,
        ],
    )
    def emb_kernel(didx_hbm, cidx_hbm, dtab_hbm, ctab_hbm, out_hbm,
                   didx_v, cidx_v, ctab_v, dr_a, dr_b,
                   gsem_a, gsem_b, osem_a, osem_b):
        cid = lax.axis_index("c")
        sid = lax.axis_index("s")
        wid = sid * NUM_CORES + cid
        idx_row0 = wid * IDX_ROWS_PER_W
        out_row0 = wid * ROWS_PER_W

        pltpu.sync_copy(didx_hbm.at[pl.ds(idx_row0, IDX_ROWS_PER_W)], didx_v)
        pltpu.sync_copy(cidx_hbm.at[pl.ds(idx_row0, IDX_ROWS_PER_W)], cidx_v)
        pltpu.sync_copy(ctab_hbm, ctab_v)

        def fire_gathers(t, drows, gsem):
            copies = []
            for j in range(JROWS):
                row = t * JROWS + j
                copies.append(pltpu.async_copy(
                    dtab_hbm.at[didx_v.at[row]],
                    drows.at[pl.ds(j * IDXV, IDXV)], gsem))
            return copies

        def alu_add(t, drows):
            for j in range(JROWS):
                idx_row = t * JROWS + j

                def body(g, carry, j=j, idx_row=idx_row):
                    c16 = cidx_v[idx_row, pl.ds(g * HALF, HALF)]
                    for k in range(HALF):
                        c = c16[k]
                        r = j * IDXV + g * HALF + k
                        plsc.addupdate(
                            drows.at[r, pl.ds(0, HALF)],
                            ctab_v[c, pl.ds(0, HALF)])
                        plsc.addupdate(
                            drows.at[r, pl.ds(HALF, HALF)],
                            ctab_v[c, pl.ds(HALF, HALF)])
                    return carry

                lax.fori_loop(0, IDXV // HALF, body, 0)

        def store(t, drows, osem):
            return pltpu.async_copy(
                drows, out_hbm.at[pl.ds(out_row0 + t * CHUNK, CHUNK)], osem)

        def pair_body(i, carry):
            t0 = 2 * i
            t1 = t0 + 1
            g_a = fire_gathers(t0, dr_a, gsem_a)
            g_b = fire_gathers(t1, dr_b, gsem_b)
            for c in g_a:
                c.wait()
            alu_add(t0, dr_a)
            o_a = store(t0, dr_a, osem_a)
            for c in g_b:
                c.wait()
            alu_add(t1, dr_b)
            o_b = store(t1, dr_b, osem_b)
            o_a.wait()
            o_b.wait()
            return carry

        lax.fori_loop(0, N_PAIRS, pair_body, 0)

    return emb_kernel


_emb_kernel = _make_kernel()


def kernel(diseases, counts, disease_table, counts_table):
    didx = diseases.astype(jnp.int32).reshape(N // IDXV, IDXV)
    cidx = counts.astype(jnp.int32).reshape(N // IDXV, IDXV)
    out = _emb_kernel(didx, cidx, disease_table, counts_table)
    return out.reshape(B, L, EMB)
